# baseline (speedup 1.0000x reference)
import numpy as np
import jax
import jax.numpy as jnp
from jax.experimental import pallas as pl
from jax.experimental.pallas import tpu as pltpu

SAMP_FREQ = 16000.0
FRAME_SHIFT_MS = 10.0
FRAME_LENGTH_MS = 25.0
PREEMPH_COEFF = 0.97
REMOVE_DC_OFFSET = True
NUM_BINS = 23
LOW_FREQ = 20.0
HIGH_FREQ = 0.0
USE_LOG_FBANK = True
USE_POWER = True
CMVN = True
NORM_VAR = False
CMVN_EPS = 1e-8

FRAME_SHIFT = int(SAMP_FREQ * 0.001 * FRAME_SHIFT_MS)
FRAME_LENGTH = int(SAMP_FREQ * 0.001 * FRAME_LENGTH_MS)


def _next_pow2(n):
    p = 1
    while p < n:
        p *= 2
    return p


NFFT = _next_pow2(FRAME_LENGTH)
K_BINS = NFFT // 2
FLT_EPS = float(np.finfo(np.float32).eps)

N_SHIFTS = -(-FRAME_LENGTH // FRAME_SHIFT)
K_PAD = N_SHIFTS * FRAME_SHIFT


def _round_up(x, m):
    return ((x + m - 1) // m) * m


def _mel_banks(num_bins, nfft, samp_freq, low_freq, high_freq):
    def mel(freq):
        return 1127.0 * np.log(1.0 + freq / 700.0)

    nyquist = 0.5 * samp_freq
    if high_freq <= 0.0:
        high_freq = nyquist + high_freq
    num_fft_bins = nfft // 2
    fft_bin_width = samp_freq / nfft
    mel_low = mel(low_freq)
    mel_high = mel(high_freq)
    delta = (mel_high - mel_low) / (num_bins + 1)
    banks = np.zeros((nfft // 2 + 1, num_bins), dtype=np.float64)
    for b in range(num_bins):
        left = mel_low + b * delta
        center = left + delta
        right = center + delta
        for i in range(num_fft_bins):
            m = mel(fft_bin_width * i)
            if left < m < right:
                if m <= center:
                    w = (m - left) / (center - left)
                else:
                    w = (right - m) / (right - center)
                banks[i, b] = w
    return banks


def _build_constants():
    L = FRAME_LENGTH
    n = np.arange(L, dtype=np.float64)
    a = 2.0 * np.pi / (L - 1)
    win = np.power(0.5 - 0.5 * np.cos(a * n), 0.85)
    P = np.eye(L, dtype=np.float64)
    P[0, 0] = 1.0 - PREEMPH_COEFF
    P[np.arange(1, L), np.arange(0, L - 1)] = -PREEMPH_COEFF
    if REMOVE_DC_OFFSET:
        D = np.eye(L, dtype=np.float64) - np.full((L, L), 1.0 / L)
    else:
        D = np.eye(L, dtype=np.float64)
    nn = np.arange(L, dtype=np.float64)[:, None]
    kk = np.arange(K_BINS, dtype=np.float64)[None, :]
    ang = 2.0 * np.pi * nn * kk / NFFT
    C = np.concatenate([np.cos(ang), -np.sin(ang)], axis=1)
    T = 32768.0 * np.diag(win) @ P @ D
    analysis_t = (C.T @ T).astype(np.float32)
    A_T = np.ascontiguousarray(analysis_t.T)
    mel = _mel_banks(NUM_BINS, NFFT, SAMP_FREQ, LOW_FREQ, HIGH_FREQ)[:K_BINS, :]
    if USE_POWER:
        mel_w = np.concatenate([mel, mel], axis=0).astype(np.float32)
    else:
        mel_w = mel.astype(np.float32)
    return A_T, mel_w


_A_T, _MEL_W = _build_constants()

CW = 640
NPH = CW // FRAME_SHIFT
_L0 = [128 * ((FRAME_SHIFT * p) // 128) for p in range(NPH)]
_OFF = [FRAME_SHIFT * p - _L0[p] for p in range(NPH)]
_N1 = [min(FRAME_LENGTH, CW - FRAME_SHIFT * p) for p in range(NPH)]
_N2 = [FRAME_LENGTH - n1 for n1 in _N1]
_WA_WIDTH = [_round_up(_OFF[p] + _N1[p], 128) for p in range(NPH)]
_WB_WIDTH = [_round_up(n2, 128) for n2 in _N2]


def _phase_weights():
    was, wbs = [], []
    for p in range(NPH):
        wa = np.zeros((_WA_WIDTH[p], 2 * K_BINS), dtype=np.float32)
        wa[_OFF[p]:_OFF[p] + _N1[p], :] = _A_T[:_N1[p], :]
        was.append(wa)
        if _N2[p] > 0:
            wb = np.zeros((_WB_WIDTH[p], 2 * K_BINS), dtype=np.float32)
            wb[:_N2[p], :] = _A_T[_N1[p]:, :]
            wbs.append(wb)
        else:
            wbs.append(None)
    return was, wbs


_WAS, _WBS = _phase_weights()


_GB = 8


def _make_body(num_valid_frames, num_rows):
    inv_n = np.float32(1.0 / float(num_valid_frames))
    F = num_valid_frames
    C = num_rows
    M = _GB * C

    def body(sig_ref, wa0, wa1, wa2, wa3, wb2, wb3, mel_ref, out_ref,
             scr_ref):
        x16 = sig_ref[...]
        z = jnp.transpose(x16.reshape(_GB, C, CW), (1, 0, 2)).reshape(M, CW)
        z = z.astype(jnp.bfloat16)
        z_next = jnp.concatenate([z[_GB:, :], z[:_GB, :]], axis=0)
        was = [wa0, wa1, wa2, wa3]
        wbs = [None, None, wb2, wb3]
        sums = []
        lms = []
        for p in range(NPH):
            y = jnp.dot(z[:, _L0[p]:_L0[p] + _WA_WIDTH[p]], was[p][...],
                        preferred_element_type=jnp.float32)
            if wbs[p] is not None:
                y += jnp.dot(z_next[:, 0:_WB_WIDTH[p]], wbs[p][...],
                             preferred_element_type=jnp.float32)
            if USE_POWER:
                pw = y * y
            else:
                re = y[:, 0:K_BINS]
                im = y[:, K_BINS:2 * K_BINS]
                pw = jnp.sqrt(re * re + im * im)
            lm = jnp.dot(pw.astype(jnp.bfloat16), mel_ref[...],
                         preferred_element_type=jnp.float32)
            if USE_LOG_FBANK:
                lm = jnp.log(jnp.maximum(lm, FLT_EPS))
            lms.append(lm)
            if CMVN:
                sm = jnp.sum(lm.reshape(C, _GB, NUM_BINS), axis=0)
                if NPH * (C - 1) + p >= F:
                    sm = sm - lm[_GB * (C - 1):, :]
                sums.append(sm)
        if CMVN:
            mean = (sums[0] + sums[1] + sums[2] + sums[3]) * inv_n
            mb = jnp.broadcast_to(mean[None, :, :],
                                  (C, _GB, NUM_BINS)).reshape(M, NUM_BINS)
        for p in range(NPH):
            lm = lms[p] - mb if CMVN else lms[p]
            scr_ref[:, _GB * p:_GB * (p + 1), :] = lm.reshape(C, _GB,
                                                             NUM_BINS)
        flat = scr_ref[...].reshape(NPH * M, NUM_BINS)
        out_ref[0] = jnp.transpose(flat)

    return body


def kernel(input_signal):
    x = input_signal.astype(jnp.float32)
    B, N = x.shape
    assert N >= FRAME_LENGTH and N % CW == 0
    F = 1 + (N - FRAME_LENGTH) // FRAME_SHIFT
    C = N // CW
    T = NPH * C
    assert F <= T - 2 and B % (2 * _GB) == 0

    xb = x

    consts = [jnp.asarray(w).astype(jnp.bfloat16) for w in
              (_WAS[0], _WAS[1], _WAS[2], _WAS[3], _WBS[2], _WBS[3],
               _MEL_W)]

    ng = B // (2 * _GB)
    feats = pl.pallas_call(
        _make_body(F, C),
        out_shape=jax.ShapeDtypeStruct((B // _GB, NUM_BINS, _GB * T),
                                       jnp.float32),
        grid=(2, ng),
        in_specs=[pl.BlockSpec((_GB, N), lambda i, g: (i * ng + g, 0))] +
                 [pl.BlockSpec(cst.shape, lambda i, g: (0, 0))
                  for cst in consts],
        out_specs=pl.BlockSpec((1, NUM_BINS, _GB * T),
                               lambda i, g: (i * ng + g, 0, 0)),
        scratch_shapes=[pltpu.VMEM((C, NPH * _GB, NUM_BINS), jnp.float32)],
        compiler_params=pltpu.CompilerParams(
            dimension_semantics=("parallel", "arbitrary")),
    )(xb, *consts)

    feats = feats.reshape(B // _GB, NUM_BINS, T, _GB)
    feats = jnp.transpose(feats, (0, 3, 1, 2)).reshape(B, NUM_BINS, T)
    return feats[:, :, :F]


# speedup vs baseline: 1.0932x; 1.0932x over previous
import numpy as np
import jax
import jax.numpy as jnp
from jax.experimental import pallas as pl
from jax.experimental.pallas import tpu as pltpu

SAMP_FREQ = 16000.0
FRAME_SHIFT_MS = 10.0
FRAME_LENGTH_MS = 25.0
PREEMPH_COEFF = 0.97
REMOVE_DC_OFFSET = True
NUM_BINS = 23
LOW_FREQ = 20.0
HIGH_FREQ = 0.0
USE_LOG_FBANK = True
USE_POWER = True
CMVN = True
NORM_VAR = False
CMVN_EPS = 1e-8

FRAME_SHIFT = int(SAMP_FREQ * 0.001 * FRAME_SHIFT_MS)
FRAME_LENGTH = int(SAMP_FREQ * 0.001 * FRAME_LENGTH_MS)


def _next_pow2(n):
    p = 1
    while p < n:
        p *= 2
    return p


NFFT = _next_pow2(FRAME_LENGTH)
K_BINS = NFFT // 2
FLT_EPS = float(np.finfo(np.float32).eps)

N_SHIFTS = -(-FRAME_LENGTH // FRAME_SHIFT)
K_PAD = N_SHIFTS * FRAME_SHIFT


def _round_up(x, m):
    return ((x + m - 1) // m) * m


def _mel_banks(num_bins, nfft, samp_freq, low_freq, high_freq):
    def mel(freq):
        return 1127.0 * np.log(1.0 + freq / 700.0)

    nyquist = 0.5 * samp_freq
    if high_freq <= 0.0:
        high_freq = nyquist + high_freq
    num_fft_bins = nfft // 2
    fft_bin_width = samp_freq / nfft
    mel_low = mel(low_freq)
    mel_high = mel(high_freq)
    delta = (mel_high - mel_low) / (num_bins + 1)
    banks = np.zeros((nfft // 2 + 1, num_bins), dtype=np.float64)
    for b in range(num_bins):
        left = mel_low + b * delta
        center = left + delta
        right = center + delta
        for i in range(num_fft_bins):
            m = mel(fft_bin_width * i)
            if left < m < right:
                if m <= center:
                    w = (m - left) / (center - left)
                else:
                    w = (right - m) / (right - center)
                banks[i, b] = w
    return banks


def _build_constants():
    L = FRAME_LENGTH
    n = np.arange(L, dtype=np.float64)
    a = 2.0 * np.pi / (L - 1)
    win = np.power(0.5 - 0.5 * np.cos(a * n), 0.85)
    P = np.eye(L, dtype=np.float64)
    P[0, 0] = 1.0 - PREEMPH_COEFF
    P[np.arange(1, L), np.arange(0, L - 1)] = -PREEMPH_COEFF
    if REMOVE_DC_OFFSET:
        D = np.eye(L, dtype=np.float64) - np.full((L, L), 1.0 / L)
    else:
        D = np.eye(L, dtype=np.float64)
    nn = np.arange(L, dtype=np.float64)[:, None]
    kk = np.arange(K_BINS, dtype=np.float64)[None, :]
    ang = 2.0 * np.pi * nn * kk / NFFT
    C = np.concatenate([np.cos(ang), -np.sin(ang)], axis=1)
    T = 32768.0 * np.diag(win) @ P @ D
    analysis_t = (C.T @ T).astype(np.float32)
    A_T = np.ascontiguousarray(analysis_t.T)
    mel = _mel_banks(NUM_BINS, NFFT, SAMP_FREQ, LOW_FREQ, HIGH_FREQ)[:K_BINS, :]
    if USE_POWER:
        mel_w = np.concatenate([mel, mel], axis=0).astype(np.float32)
    else:
        mel_w = mel.astype(np.float32)
    return A_T, mel_w


_A_T, _MEL_W = _build_constants()

CW = 640
NPH = CW // FRAME_SHIFT
_L0 = [128 * ((FRAME_SHIFT * p) // 128) for p in range(NPH)]
_OFF = [FRAME_SHIFT * p - _L0[p] for p in range(NPH)]
_N1 = [min(FRAME_LENGTH, CW - FRAME_SHIFT * p) for p in range(NPH)]
_N2 = [FRAME_LENGTH - n1 for n1 in _N1]
_WA_WIDTH = [_round_up(_OFF[p] + _N1[p], 128) for p in range(NPH)]
_WB_WIDTH = [_round_up(n2, 128) for n2 in _N2]


def _phase_weights():
    was, wbs = [], []
    for p in range(NPH):
        wa = np.zeros((_WA_WIDTH[p], 2 * K_BINS), dtype=np.float32)
        wa[_OFF[p]:_OFF[p] + _N1[p], :] = _A_T[:_N1[p], :]
        was.append(wa)
        if _N2[p] > 0:
            wb = np.zeros((_WB_WIDTH[p], 2 * K_BINS), dtype=np.float32)
            wb[:_N2[p], :] = _A_T[_N1[p]:, :]
            wbs.append(wb)
        else:
            wbs.append(None)
    return was, wbs


_WAS, _WBS = _phase_weights()


_PLANES = 8
_CP = 256


def _make_body(num_valid_frames, num_rows):
    inv_n = np.float32(1.0 / float(num_valid_frames))
    F = num_valid_frames
    C = num_rows

    def body(sig_ref, wa0, wa1, wa2, wa3, wb2, wb3, mel_ref, out_ref,
             scr_ref):
        zpad = jnp.zeros((_CP - C, CW), dtype=sig_ref.dtype)
        s = jnp.concatenate(
            sum([[sig_ref[k], zpad] for k in range(_PLANES)], []), axis=0)
        s_next = jnp.concatenate([s[1:, :], s[:1, :]], axis=0)
        was = [wa0, wa1, wa2, wa3]
        wbs = [None, None, wb2, wb3]
        for p in range(NPH):
            y = jnp.dot(s[:, _L0[p]:_L0[p] + _WA_WIDTH[p]], was[p][...],
                        preferred_element_type=jnp.float32)
            if wbs[p] is not None:
                y += jnp.dot(s_next[:, 0:_WB_WIDTH[p]], wbs[p][...],
                             preferred_element_type=jnp.float32)
            if USE_POWER:
                pw = y * y
            else:
                re = y[:, 0:K_BINS]
                im = y[:, K_BINS:2 * K_BINS]
                pw = jnp.sqrt(re * re + im * im)
            lm = jnp.dot(pw.astype(jnp.bfloat16), mel_ref[...],
                         preferred_element_type=jnp.float32)
            if USE_LOG_FBANK:
                lm = jnp.log(jnp.maximum(lm, FLT_EPS))
            for k in range(_PLANES):
                scr_ref[NPH * C * k + p:NPH * C * (k + 1):NPH, :] = \
                    lm[_CP * k:_CP * k + C, :]
        for k in range(_PLANES):
            lm_t = jnp.transpose(
                scr_ref[NPH * C * k:NPH * C * (k + 1), :])[:, 0:F]
            if CMVN:
                mean = jnp.sum(lm_t, axis=1, keepdims=True) * inv_n
                centered = lm_t - mean
                if NORM_VAR:
                    var = jnp.sum(centered * centered, axis=1,
                                  keepdims=True) * inv_n
                    centered = centered * jax.lax.rsqrt(var + CMVN_EPS)
                out_ref[k] = centered
            else:
                out_ref[k] = lm_t

    return body


def kernel(input_signal):
    x = input_signal.astype(jnp.float32)
    B, N = x.shape
    assert N >= FRAME_LENGTH and N % CW == 0
    F = 1 + (N - FRAME_LENGTH) // FRAME_SHIFT
    C = N // CW
    assert F <= NPH * C - 2 and B % 2 == 0

    chunks = x.reshape(B, C, CW).astype(jnp.bfloat16)

    consts = [jnp.asarray(w).astype(jnp.bfloat16) for w in
              (_WAS[0], _WAS[1], _WAS[2], _WAS[3], _WBS[2], _WBS[3],
               _MEL_W)]

    assert B % (2 * _PLANES) == 0 and C <= _CP
    nb = B // (2 * _PLANES)
    return pl.pallas_call(
        _make_body(F, C),
        out_shape=jax.ShapeDtypeStruct((B, NUM_BINS, F), jnp.float32),
        grid=(2, nb),
        in_specs=[pl.BlockSpec((_PLANES, C, CW),
                               lambda i, b: (i * nb + b, 0, 0))] +
                 [pl.BlockSpec(cst.shape, lambda i, b: (0, 0))
                  for cst in consts],
        out_specs=pl.BlockSpec((_PLANES, NUM_BINS, F),
                               lambda i, b: (i * nb + b, 0, 0)),
        scratch_shapes=[pltpu.VMEM((_PLANES * NPH * C, NUM_BINS),
                                   jnp.float32)],
        compiler_params=pltpu.CompilerParams(
            dimension_semantics=("parallel", "arbitrary")),
    )(chunks, *consts)


# speedup vs baseline: 1.2145x; 1.1110x over previous
import numpy as np
import jax
import jax.numpy as jnp
from jax.experimental import pallas as pl
from jax.experimental.pallas import tpu as pltpu

SAMP_FREQ = 16000.0
FRAME_SHIFT_MS = 10.0
FRAME_LENGTH_MS = 25.0
PREEMPH_COEFF = 0.97
REMOVE_DC_OFFSET = True
NUM_BINS = 23
LOW_FREQ = 20.0
HIGH_FREQ = 0.0
USE_LOG_FBANK = True
USE_POWER = True
CMVN = True
NORM_VAR = False
CMVN_EPS = 1e-8

FRAME_SHIFT = int(SAMP_FREQ * 0.001 * FRAME_SHIFT_MS)
FRAME_LENGTH = int(SAMP_FREQ * 0.001 * FRAME_LENGTH_MS)


def _next_pow2(n):
    p = 1
    while p < n:
        p *= 2
    return p


NFFT = _next_pow2(FRAME_LENGTH)
K_BINS = NFFT // 2
FLT_EPS = float(np.finfo(np.float32).eps)

N_SHIFTS = -(-FRAME_LENGTH // FRAME_SHIFT)
K_PAD = N_SHIFTS * FRAME_SHIFT


def _round_up(x, m):
    return ((x + m - 1) // m) * m


def _mel_banks(num_bins, nfft, samp_freq, low_freq, high_freq):
    def mel(freq):
        return 1127.0 * np.log(1.0 + freq / 700.0)

    nyquist = 0.5 * samp_freq
    if high_freq <= 0.0:
        high_freq = nyquist + high_freq
    num_fft_bins = nfft // 2
    fft_bin_width = samp_freq / nfft
    mel_low = mel(low_freq)
    mel_high = mel(high_freq)
    delta = (mel_high - mel_low) / (num_bins + 1)
    banks = np.zeros((nfft // 2 + 1, num_bins), dtype=np.float64)
    for b in range(num_bins):
        left = mel_low + b * delta
        center = left + delta
        right = center + delta
        for i in range(num_fft_bins):
            m = mel(fft_bin_width * i)
            if left < m < right:
                if m <= center:
                    w = (m - left) / (center - left)
                else:
                    w = (right - m) / (right - center)
                banks[i, b] = w
    return banks


def _build_constants():
    L = FRAME_LENGTH
    n = np.arange(L, dtype=np.float64)
    a = 2.0 * np.pi / (L - 1)
    win = np.power(0.5 - 0.5 * np.cos(a * n), 0.85)
    P = np.eye(L, dtype=np.float64)
    P[0, 0] = 1.0 - PREEMPH_COEFF
    P[np.arange(1, L), np.arange(0, L - 1)] = -PREEMPH_COEFF
    if REMOVE_DC_OFFSET:
        D = np.eye(L, dtype=np.float64) - np.full((L, L), 1.0 / L)
    else:
        D = np.eye(L, dtype=np.float64)
    nn = np.arange(L, dtype=np.float64)[:, None]
    kk = np.arange(K_BINS, dtype=np.float64)[None, :]
    ang = 2.0 * np.pi * nn * kk / NFFT
    C = np.concatenate([np.cos(ang), -np.sin(ang)], axis=1)
    T = 32768.0 * np.diag(win) @ P @ D
    analysis_t = (C.T @ T).astype(np.float32)
    A_T = np.ascontiguousarray(analysis_t.T)
    mel = _mel_banks(NUM_BINS, NFFT, SAMP_FREQ, LOW_FREQ, HIGH_FREQ)[:K_BINS, :]
    if USE_POWER:
        mel_w = np.concatenate([mel, mel], axis=0).astype(np.float32)
    else:
        mel_w = mel.astype(np.float32)
    return A_T, mel_w


_A_T, _MEL_W = _build_constants()

CW = 640
NPH = CW // FRAME_SHIFT
_L0 = [128 * ((FRAME_SHIFT * p) // 128) for p in range(NPH)]
_OFF = [FRAME_SHIFT * p - _L0[p] for p in range(NPH)]
_N1 = [min(FRAME_LENGTH, CW - FRAME_SHIFT * p) for p in range(NPH)]
_N2 = [FRAME_LENGTH - n1 for n1 in _N1]
_WA_WIDTH = [_round_up(_OFF[p] + _N1[p], 128) for p in range(NPH)]
_WB_WIDTH = [_round_up(n2, 128) for n2 in _N2]


def _phase_weights():
    was, wbs = [], []
    for p in range(NPH):
        wa = np.zeros((_WA_WIDTH[p], 2 * K_BINS), dtype=np.float32)
        wa[_OFF[p]:_OFF[p] + _N1[p], :] = _A_T[:_N1[p], :]
        was.append(wa)
        if _N2[p] > 0:
            wb = np.zeros((_WB_WIDTH[p], 2 * K_BINS), dtype=np.float32)
            wb[:_N2[p], :] = _A_T[_N1[p]:, :]
            wbs.append(wb)
        else:
            wbs.append(None)
    return was, wbs


_WAS, _WBS = _phase_weights()


_PLANES = 8
_CP = 256


def _make_body(num_valid_frames, num_rows):
    inv_n = np.float32(1.0 / float(num_valid_frames))
    F = num_valid_frames
    C = num_rows

    def body(sig_ref, wa0, wa1, wa2, wa3, wb2, wb3, mel_ref, out_ref,
             scr_ref):
        zpad = jnp.zeros((_CP - C, CW), dtype=sig_ref.dtype)
        s = jnp.concatenate(
            sum([[sig_ref[k], zpad] for k in range(_PLANES)], []),
            axis=0).astype(jnp.bfloat16)
        s_next = jnp.concatenate([s[1:, :], s[:1, :]], axis=0)
        was = [wa0, wa1, wa2, wa3]
        wbs = [None, None, wb2, wb3]
        for p in range(NPH):
            y = jnp.dot(s[:, _L0[p]:_L0[p] + _WA_WIDTH[p]], was[p][...],
                        preferred_element_type=jnp.float32)
            if wbs[p] is not None:
                y += jnp.dot(s_next[:, 0:_WB_WIDTH[p]], wbs[p][...],
                             preferred_element_type=jnp.float32)
            if USE_POWER:
                pw = y * y
            else:
                re = y[:, 0:K_BINS]
                im = y[:, K_BINS:2 * K_BINS]
                pw = jnp.sqrt(re * re + im * im)
            lm = jnp.dot(pw.astype(jnp.bfloat16), mel_ref[...],
                         preferred_element_type=jnp.float32)
            if USE_LOG_FBANK:
                lm = jnp.log(jnp.maximum(lm, FLT_EPS))
            for k in range(_PLANES):
                scr_ref[NPH * C * k + p:NPH * C * (k + 1):NPH, :] = \
                    lm[_CP * k:_CP * k + C, :]
        for k in range(_PLANES):
            lm_t = jnp.transpose(
                scr_ref[NPH * C * k:NPH * C * (k + 1), :])[:, 0:F]
            if CMVN:
                mean = jnp.sum(lm_t, axis=1, keepdims=True) * inv_n
                centered = lm_t - mean
                if NORM_VAR:
                    var = jnp.sum(centered * centered, axis=1,
                                  keepdims=True) * inv_n
                    centered = centered * jax.lax.rsqrt(var + CMVN_EPS)
                out_ref[k] = centered
            else:
                out_ref[k] = lm_t

    return body


def kernel(input_signal):
    x = input_signal.astype(jnp.float32)
    B, N = x.shape
    assert N >= FRAME_LENGTH and N % CW == 0
    F = 1 + (N - FRAME_LENGTH) // FRAME_SHIFT
    C = N // CW
    assert F <= NPH * C - 2 and B % 2 == 0

    chunks = x.reshape(B, C, CW)

    consts = [jnp.asarray(w).astype(jnp.bfloat16) for w in
              (_WAS[0], _WAS[1], _WAS[2], _WAS[3], _WBS[2], _WBS[3],
               _MEL_W)]

    assert B % (2 * _PLANES) == 0 and C <= _CP
    nb = B // (2 * _PLANES)
    return pl.pallas_call(
        _make_body(F, C),
        out_shape=jax.ShapeDtypeStruct((B, NUM_BINS, F), jnp.float32),
        grid=(2, nb),
        in_specs=[pl.BlockSpec((_PLANES, C, CW),
                               lambda i, b: (i * nb + b, 0, 0))] +
                 [pl.BlockSpec(cst.shape, lambda i, b: (0, 0))
                  for cst in consts],
        out_specs=pl.BlockSpec((_PLANES, NUM_BINS, F),
                               lambda i, b: (i * nb + b, 0, 0)),
        scratch_shapes=[pltpu.VMEM((_PLANES * NPH * C, NUM_BINS),
                                   jnp.float32)],
        compiler_params=pltpu.CompilerParams(
            dimension_semantics=("parallel", "arbitrary")),
    )(chunks, *consts)
